# Initial kernel scaffold; baseline (speedup 1.0000x reference)
import jax
import jax.numpy as jnp
from jax.experimental import pallas as pl


def kernel(hidden_states, weight):
    raise NotImplementedError("write your pallas kernel here")


# speedup vs baseline: 1.0769x; 1.0769x over previous
import functools

import jax
import jax.numpy as jnp
from jax.experimental import pallas as pl
from jax.experimental.pallas import tpu as pltpu

HID = 4096
NE = 64
TOPK = 8
BLK = 512


def _gate_kernel(x_ref, w_ref, wout_ref, iout_ref):
    x = x_ref[...]
    w = w_ref[...]
    logits = jax.lax.dot_general(
        x, w, (((1,), (1,)), ((), ())),
        preferred_element_type=jnp.float32)
    iota = jax.lax.broadcasted_iota(jnp.int32, (BLK, NE), 1)
    s = logits
    vals = []
    idxs = []
    for _ in range(TOPK):
        m = jnp.max(s, axis=1, keepdims=True)
        idx = jnp.min(jnp.where(s == m, iota, NE), axis=1,
                      keepdims=True)
        vals.append(m)
        idxs.append(idx)
        s = jnp.where(iota == idx, -jnp.inf, s)
    v = jnp.concatenate(vals, axis=1)
    i = jnp.concatenate(idxs, axis=1)
    e = jnp.exp(v - v[:, :1])
    wout_ref[...] = e / jnp.sum(e, axis=1, keepdims=True)
    iout_ref[...] = i


@jax.jit
def kernel(hidden_states, weight):
    b, s, h = hidden_states.shape
    n = b * s
    x = hidden_states.reshape(n, h).astype(jnp.float32)
    grid = (n // BLK,)
    wout, iout = pl.pallas_call(
        _gate_kernel,
        grid=grid,
        in_specs=[
            pl.BlockSpec((BLK, HID), lambda t: (t, 0)),
            pl.BlockSpec((NE, HID), lambda t: (0, 0)),
        ],
        out_specs=[
            pl.BlockSpec((BLK, TOPK), lambda t: (t, 0)),
            pl.BlockSpec((BLK, TOPK), lambda t: (t, 0)),
        ],
        out_shape=[
            jax.ShapeDtypeStruct((n, TOPK), jnp.float32),
            jax.ShapeDtypeStruct((n, TOPK), jnp.int32),
        ],
        compiler_params=pltpu.CompilerParams(
            dimension_semantics=("arbitrary",)),
    )(x, weight.astype(jnp.float32))
    return wout, iout


# speedup vs baseline: 1.1627x; 1.0797x over previous
import functools

import jax
import jax.numpy as jnp
from jax import lax
from jax.experimental import pallas as pl
from jax.experimental.pallas import tpu as pltpu
from jax.experimental.pallas import tpu_sc as plsc

HID = 4096
NE = 64
TOPK = 8
BLK = 512
NC, NS, L = 2, 16, 16
NW = NC * NS
TPW = 8192 // NW
NEG = -3.0e38


def _matmul_kernel(x_ref, w_ref, out_ref):
    logits = lax.dot_general(
        w_ref[...], x_ref[...], (((1,), (1,)), ((), ())),
        preferred_element_type=jnp.float32)
    for c in range(BLK // TPW):
        out_ref[c] = logits[:, c * TPW:(c + 1) * TPW]


_sc_mesh = plsc.VectorSubcoreMesh(core_axis_name="c", subcore_axis_name="s")


@functools.partial(
    pl.kernel,
    mesh=_sc_mesh,
    out_type=[
        jax.ShapeDtypeStruct((NW, TOPK, TPW), jnp.float32),
        jax.ShapeDtypeStruct((NW, TOPK, TPW), jnp.int32),
    ],
    scratch_types=[
        pltpu.VMEM((NE, TPW), jnp.float32),
        pltpu.VMEM((TOPK, TPW), jnp.float32),
        pltpu.VMEM((TOPK, TPW), jnp.int32),
    ],
)
def _topk_kernel(lg_hbm, wout_hbm, iout_hbm, lg_v, wv, iv):
    wid = lax.axis_index("s") * NC + lax.axis_index("c")
    pltpu.sync_copy(lg_hbm.at[wid], lg_v)

    def group(g, carry):
        off = g * L
        vals = [jnp.full((L,), NEG, jnp.float32) for _ in range(TOPK)]
        idxs = [jnp.zeros((L,), jnp.int32) for _ in range(TOPK)]
        for e in range(NE):
            v = lg_v[e, pl.ds(off, L)]
            ev = jnp.full((L,), e, jnp.int32)
            c = v > vals[TOPK - 1]
            vals[TOPK - 1] = jnp.where(c, v, vals[TOPK - 1])
            idxs[TOPK - 1] = jnp.where(c, ev, idxs[TOPK - 1])
            for j in range(TOPK - 1, 0, -1):
                c2 = vals[j] > vals[j - 1]
                vhi = jnp.maximum(vals[j - 1], vals[j])
                vlo = jnp.minimum(vals[j - 1], vals[j])
                ihi = jnp.where(c2, idxs[j], idxs[j - 1])
                ilo = jnp.where(c2, idxs[j - 1], idxs[j])
                vals[j - 1], vals[j] = vhi, vlo
                idxs[j - 1], idxs[j] = ihi, ilo
        es = [jnp.exp(v - vals[0]) for v in vals]
        s = es[0]
        for j in range(1, TOPK):
            s = s + es[j]
        for j in range(TOPK):
            wv[j, pl.ds(off, L)] = es[j] / s
            iv[j, pl.ds(off, L)] = idxs[j]
        return carry

    lax.fori_loop(0, TPW // L, group, 0)
    pltpu.sync_copy(wv, wout_hbm.at[wid])
    pltpu.sync_copy(iv, iout_hbm.at[wid])


@jax.jit
def kernel(hidden_states, weight):
    b, s, h = hidden_states.shape
    n = b * s
    x = hidden_states.reshape(n, h).astype(jnp.float32)
    lg = pl.pallas_call(
        _matmul_kernel,
        grid=(n // BLK,),
        in_specs=[
            pl.BlockSpec((BLK, HID), lambda t: (t, 0)),
            pl.BlockSpec((NE, HID), lambda t: (0, 0)),
        ],
        out_specs=pl.BlockSpec((BLK // TPW, NE, TPW), lambda t: (t, 0, 0)),
        out_shape=jax.ShapeDtypeStruct((n // TPW, NE, TPW), jnp.float32),
        compiler_params=pltpu.CompilerParams(
            dimension_semantics=("arbitrary",)),
    )(x, weight.astype(jnp.float32))
    wt, it = _topk_kernel(lg)
    wout = wt.transpose(0, 2, 1).reshape(n, TOPK)
    iout = it.transpose(0, 2, 1).reshape(n, TOPK)
    return wout, iout
